# baseline (speedup 1.0000x reference)
import functools

import jax
import jax.numpy as jnp
from jax import lax
from jax.experimental import pallas as pl
from jax.experimental.pallas import tpu as pltpu
from jax.experimental.pallas import tpu_sc as plsc

B = 16384
D = 128
H = 16
NC = 2
NS = 16
NW = NC * NS
BPW = B // NW
CHUNK = 128
NCH = BPW // CHUNK


def _gather_body(idx_hbm, tab, out, idx_v, *bufsem):
    rows = bufsem[:NCH]
    gsem = bufsem[NCH:2 * NCH]
    ssem = bufsem[2 * NCH:]
    wid = lax.axis_index("s") * NC + lax.axis_index("c")
    base = wid * BPW
    pltpu.sync_copy(idx_hbm.at[pl.ds(NCH * wid, NCH)], idx_v)
    for j in range(NCH):
        pltpu.async_copy(tab.at[idx_v.at[j]], rows[j], gsem[j])
    for j in range(NCH):
        pltpu.make_async_copy(tab.at[idx_v.at[j]], rows[j], gsem[j]).wait()
        pltpu.async_copy(rows[j], out.at[pl.ds(base + j * CHUNK, CHUNK)],
                         ssem[j])
    for j in range(NCH):
        pltpu.make_async_copy(rows[j],
                              out.at[pl.ds(base + j * CHUNK, CHUNK)],
                              ssem[j]).wait()


@functools.cache
def _gather():
    return functools.partial(
        pl.kernel,
        mesh=plsc.VectorSubcoreMesh(core_axis_name="c", subcore_axis_name="s"),
        out_type=jax.ShapeDtypeStruct((B, D), jnp.float32),
        scratch_types=(
            [pltpu.VMEM((NCH, CHUNK), jnp.int32)]
            + [pltpu.VMEM((CHUNK, D), jnp.float32)] * NCH
            + [pltpu.SemaphoreType.DMA] * (2 * NCH)
        ),
    )(_gather_body)


BM = 4096


def _proj_body(u_ref, w1_ref, b1_ref, o_ref):
    hT = lax.dot_general(
        w1_ref[:D, :], u_ref[...], (((0,), (1,)), ((), ())),
        preferred_element_type=jnp.float32,
    )
    o_ref[...] = hT + b1_ref[...]


def _proj(u_rows, W1, b1):
    return pl.pallas_call(
        _proj_body,
        grid=(B // BM,),
        in_specs=[
            pl.BlockSpec((BM, D), lambda i: (i, 0)),
            pl.BlockSpec((2 * D, H), lambda i: (0, 0)),
            pl.BlockSpec((H, 1), lambda i: (0, 0)),
        ],
        out_specs=pl.BlockSpec((H, BM), lambda i: (0, i)),
        out_shape=jax.ShapeDtypeStruct((H, B), jnp.float32),
    )(u_rows, W1, b1)


def _finish_body(hu_ref, p_ref, w1_ref, w2_ref, b2_ref, o_ref):
    hT = hu_ref[...] + lax.dot_general(
        w1_ref[D:, :], p_ref[...], (((0,), (1,)), ((), ())),
        preferred_element_type=jnp.float32,
    )
    r = jnp.maximum(hT, 0.0) * w2_ref[...]
    o_ref[...] = jnp.sum(r, axis=0) + b2_ref[0, 0]


def _finish(h_uT, p_rows, W1, W2, b2):
    return pl.pallas_call(
        _finish_body,
        grid=(B // BM,),
        in_specs=[
            pl.BlockSpec((H, BM), lambda i: (0, i)),
            pl.BlockSpec((BM, D), lambda i: (i, 0)),
            pl.BlockSpec((2 * D, H), lambda i: (0, 0)),
            pl.BlockSpec((H, 1), lambda i: (0, 0)),
            pl.BlockSpec((1, 1), lambda i: (0, 0)),
        ],
        out_specs=pl.BlockSpec((BM,), lambda i: (i,)),
        out_shape=jax.ShapeDtypeStruct((B,), jnp.float32),
    )(h_uT, p_rows, W1, W2, b2)


def kernel(inputs, user_table, prod_table, W1, b1, W2, b2):
    uidx = inputs[:, 0].astype(jnp.int32).reshape(B // CHUNK, CHUNK)
    pidx = inputs[:, 1].astype(jnp.int32).reshape(B // CHUNK, CHUNK)
    gather = _gather()
    u_rows = gather(uidx, user_table)
    p_rows = gather(pidx, prod_table)
    h_uT = _proj(u_rows, W1, b1.reshape(H, 1))
    out = _finish(h_uT, p_rows, W1, W2.reshape(H, 1), b2.reshape(1, 1))
    return out.reshape(B, 1)


# speedup vs baseline: 1.0061x; 1.0061x over previous
import functools

import jax
import jax.numpy as jnp
from jax import lax
from jax.experimental import pallas as pl
from jax.experimental.pallas import tpu as pltpu
from jax.experimental.pallas import tpu_sc as plsc

B = 16384
D = 128
H = 16
NC = 2
NS = 16
NW = NC * NS
BPW = B // NW
CHUNK = 64
NCH = BPW // CHUNK


def _gather_body(idx_hbm, tab, out, idx_v, *bufsem):
    rows = bufsem[:NCH]
    gsem = bufsem[NCH:2 * NCH]
    ssem = bufsem[2 * NCH:]
    wid = lax.axis_index("s") * NC + lax.axis_index("c")
    base = wid * BPW
    pltpu.sync_copy(idx_hbm.at[pl.ds(NCH * wid, NCH)], idx_v)
    for j in range(NCH):
        pltpu.async_copy(tab.at[idx_v.at[j]], rows[j], gsem[j])
    for j in range(NCH):
        pltpu.make_async_copy(tab.at[idx_v.at[j]], rows[j], gsem[j]).wait()
        pltpu.async_copy(rows[j], out.at[pl.ds(base + j * CHUNK, CHUNK)],
                         ssem[j])
    for j in range(NCH):
        pltpu.make_async_copy(rows[j],
                              out.at[pl.ds(base + j * CHUNK, CHUNK)],
                              ssem[j]).wait()


@functools.cache
def _gather():
    return functools.partial(
        pl.kernel,
        mesh=plsc.VectorSubcoreMesh(core_axis_name="c", subcore_axis_name="s"),
        out_type=jax.ShapeDtypeStruct((B, D), jnp.float32),
        scratch_types=(
            [pltpu.VMEM((NCH, CHUNK), jnp.int32)]
            + [pltpu.VMEM((CHUNK, D), jnp.float32)] * NCH
            + [pltpu.SemaphoreType.DMA] * (2 * NCH)
        ),
    )(_gather_body)


BM = 4096


def _proj_body(u_ref, w1_ref, b1_ref, o_ref):
    hT = lax.dot_general(
        w1_ref[:D, :], u_ref[...], (((0,), (1,)), ((), ())),
        preferred_element_type=jnp.float32,
    )
    o_ref[...] = hT + b1_ref[...]


def _proj(u_rows, W1, b1):
    return pl.pallas_call(
        _proj_body,
        grid=(B // BM,),
        in_specs=[
            pl.BlockSpec((BM, D), lambda i: (i, 0)),
            pl.BlockSpec((2 * D, H), lambda i: (0, 0)),
            pl.BlockSpec((H, 1), lambda i: (0, 0)),
        ],
        out_specs=pl.BlockSpec((H, BM), lambda i: (0, i)),
        out_shape=jax.ShapeDtypeStruct((H, B), jnp.float32),
    )(u_rows, W1, b1)


def _finish_body(hu_ref, p_ref, w1_ref, w2_ref, b2_ref, o_ref):
    hT = hu_ref[...] + lax.dot_general(
        w1_ref[D:, :], p_ref[...], (((0,), (1,)), ((), ())),
        preferred_element_type=jnp.float32,
    )
    r = jnp.maximum(hT, 0.0) * w2_ref[...]
    o_ref[...] = jnp.sum(r, axis=0) + b2_ref[0, 0]


def _finish(h_uT, p_rows, W1, W2, b2):
    return pl.pallas_call(
        _finish_body,
        grid=(B // BM,),
        in_specs=[
            pl.BlockSpec((H, BM), lambda i: (0, i)),
            pl.BlockSpec((BM, D), lambda i: (i, 0)),
            pl.BlockSpec((2 * D, H), lambda i: (0, 0)),
            pl.BlockSpec((H, 1), lambda i: (0, 0)),
            pl.BlockSpec((1, 1), lambda i: (0, 0)),
        ],
        out_specs=pl.BlockSpec((BM,), lambda i: (i,)),
        out_shape=jax.ShapeDtypeStruct((B,), jnp.float32),
    )(h_uT, p_rows, W1, W2, b2)


def kernel(inputs, user_table, prod_table, W1, b1, W2, b2):
    uidx = inputs[:, 0].astype(jnp.int32).reshape(B // CHUNK, CHUNK)
    pidx = inputs[:, 1].astype(jnp.int32).reshape(B // CHUNK, CHUNK)
    gather = _gather()
    u_rows = gather(uidx, user_table)
    p_rows = gather(pidx, prod_table)
    h_uT = _proj(u_rows, W1, b1.reshape(H, 1))
    out = _finish(h_uT, p_rows, W1, W2.reshape(H, 1), b2.reshape(1, 1))
    return out.reshape(B, 1)
